# baseline (speedup 1.0000x reference)
import functools

import jax
import jax.numpy as jnp
from jax import lax
from jax.experimental import pallas as pl
from jax.experimental.pallas import tpu as pltpu
from jax.experimental.pallas import tpu_sc as plsc

DIM = 64
NUM_CORES = 2
NUM_SUBCORES = 16
NUM_WORKERS = NUM_CORES * NUM_SUBCORES
SUB = 128
GATHERS = 8
CHUNK = SUB * GATHERS


@jax.jit
def _embed_sc(xf, table):
    B = xf.shape[0]
    b_per_w = B // NUM_WORKERS
    n_chunks = b_per_w // CHUNK

    mesh = plsc.VectorSubcoreMesh(core_axis_name="c", subcore_axis_name="s")

    @functools.partial(
        pl.kernel,
        mesh=mesh,
        compiler_params=pltpu.CompilerParams(use_tc_tiling_on_sc=False),
        out_type=jax.ShapeDtypeStruct((B, DIM), jnp.float32),
        scratch_types=[
            pltpu.VMEM((CHUNK,), jnp.int32),
            pltpu.VMEM((CHUNK, DIM), jnp.float32),
            pltpu.SemaphoreType.DMA,
        ],
    )
    def emb(x_hbm, tab_hbm, out_hbm, idx_v, rows_v, sem):
        wid = lax.axis_index("s") * NUM_CORES + lax.axis_index("c")
        base = wid * b_per_w

        def body(g, carry):
            off = base + g * CHUNK
            pltpu.sync_copy(x_hbm.at[pl.ds(off, CHUNK)], idx_v)
            copies = []
            for j in range(GATHERS):
                copies.append(
                    pltpu.async_copy(
                        tab_hbm.at[idx_v.at[pl.ds(j * SUB, SUB)]],
                        rows_v.at[pl.ds(j * SUB, SUB), :],
                        sem,
                    )
                )
            for c in copies:
                c.wait()
            pltpu.sync_copy(rows_v, out_hbm.at[pl.ds(off, CHUNK)])
            return carry

        lax.fori_loop(0, n_chunks, body, 0)

    return emb(xf, table)


def kernel(x, table):
    xf = x.reshape(-1).astype(jnp.int32)
    out = _embed_sc(xf, table)
    return out.reshape(x.shape + (DIM,))


# speedup vs baseline: 1.0162x; 1.0162x over previous
import functools

import jax
import jax.numpy as jnp
from jax import lax
from jax.experimental import pallas as pl
from jax.experimental.pallas import tpu as pltpu
from jax.experimental.pallas import tpu_sc as plsc

DIM = 64
NUM_CORES = 2
NUM_SUBCORES = 16
NUM_WORKERS = NUM_CORES * NUM_SUBCORES
SUB = 128
GATHERS = 5
CHUNK = SUB * GATHERS
NBUF = 2


@jax.jit
def _embed_sc(xf, table):
    B = xf.shape[0]
    b_per_w = B // NUM_WORKERS
    n_chunks = b_per_w // CHUNK
    n_rounds = n_chunks // NBUF
    assert b_per_w % CHUNK == 0 and n_chunks % NBUF == 0

    mesh = plsc.VectorSubcoreMesh(core_axis_name="c", subcore_axis_name="s")

    @functools.partial(
        pl.kernel,
        mesh=mesh,
        compiler_params=pltpu.CompilerParams(use_tc_tiling_on_sc=False),
        out_type=jax.ShapeDtypeStruct((B, DIM), jnp.float32),
        scratch_types=[
            pltpu.VMEM((b_per_w,), jnp.int32),
            pltpu.VMEM((NBUF, CHUNK, DIM), jnp.float32),
            pltpu.SemaphoreType.DMA,
            pltpu.SemaphoreType.DMA,
            pltpu.SemaphoreType.DMA,
            pltpu.SemaphoreType.DMA,
        ],
    )
    def emb(x_hbm, tab_hbm, out_hbm, idx_v, rows_v, sg0, sg1, sw0, sw1):
        sg = [sg0, sg1]
        sw = [sw0, sw1]
        wid = lax.axis_index("s") * NUM_CORES + lax.axis_index("c")
        base = wid * b_per_w

        pltpu.sync_copy(x_hbm.at[pl.ds(base, b_per_w)], idx_v)

        def fire_gathers(c, b):
            for j in range(GATHERS):
                pltpu.async_copy(
                    tab_hbm.at[idx_v.at[pl.ds(c * CHUNK + j * SUB, SUB)]],
                    rows_v.at[b].at[pl.ds(j * SUB, SUB), :],
                    sg[b],
                )

        def drain_gathers(b):
            pltpu.make_async_copy(
                out_hbm.at[pl.ds(0, CHUNK)], rows_v.at[b], sg[b]
            ).wait()

        def drain_write(b):
            pltpu.make_async_copy(
                out_hbm.at[pl.ds(0, CHUNK)], rows_v.at[b], sw[b]
            ).wait()

        for b in range(NBUF - 1):
            fire_gathers(b, b)

        def round_body(r, carry):
            for b in range(NBUF):
                c = r * NBUF + b
                bf = (b + NBUF - 1) % NBUF
                f = c + NBUF - 1

                @pl.when(c >= 1)
                def _():
                    drain_write(bf)

                @pl.when(f < n_chunks)
                def _():
                    fire_gathers(f, bf)

                drain_gathers(b)
                pltpu.async_copy(
                    rows_v.at[b],
                    out_hbm.at[pl.ds(base + c * CHUNK, CHUNK)],
                    sw[b],
                )
            return carry

        lax.fori_loop(0, n_rounds, round_body, 0)
        drain_write((n_chunks - 1) % NBUF)

    return emb(xf, table)


def kernel(x, table):
    xf = x.reshape(-1).astype(jnp.int32)
    out = _embed_sc(xf, table)
    return out.reshape(x.shape + (DIM,))


# speedup vs baseline: 1.2400x; 1.2202x over previous
import functools

import jax
import jax.numpy as jnp
from jax import lax
from jax.experimental import pallas as pl
from jax.experimental.pallas import tpu as pltpu
from jax.experimental.pallas import tpu_sc as plsc

DIM = 64
PDIM = 128
NUM_CORES = 2
NUM_SUBCORES = 16
NUM_WORKERS = NUM_CORES * NUM_SUBCORES
SUB = 64
GATHERS = 5
CHUNK = SUB * GATHERS
NBUF = 2


@jax.jit
def _embed_sc(xf, tpad):
    B = xf.shape[0]
    b_per_w = B // NUM_WORKERS
    n_chunks = b_per_w // CHUNK
    n_rounds = n_chunks // NBUF
    assert b_per_w % CHUNK == 0 and n_chunks % NBUF == 0

    mesh = plsc.VectorSubcoreMesh(core_axis_name="c", subcore_axis_name="s")

    @functools.partial(
        pl.kernel,
        mesh=mesh,
        compiler_params=pltpu.CompilerParams(use_tc_tiling_on_sc=True),
        out_type=jax.ShapeDtypeStruct((B, PDIM), jnp.float32),
        scratch_types=[
            pltpu.VMEM((b_per_w,), jnp.int32),
            pltpu.VMEM((NBUF, CHUNK, PDIM), jnp.float32),
            pltpu.SemaphoreType.DMA,
            pltpu.SemaphoreType.DMA,
            pltpu.SemaphoreType.DMA,
            pltpu.SemaphoreType.DMA,
        ],
    )
    def emb(x_hbm, tab_hbm, out_hbm, idx_v, rows_v, sg0, sg1, sw0, sw1):
        sg = [sg0, sg1]
        sw = [sw0, sw1]
        wid = lax.axis_index("s") * NUM_CORES + lax.axis_index("c")
        base = wid * b_per_w

        pltpu.sync_copy(x_hbm.at[pl.ds(base, b_per_w)], idx_v)

        def fire_gathers(c, b):
            for j in range(GATHERS):
                pltpu.async_copy(
                    tab_hbm.at[idx_v.at[pl.ds(c * CHUNK + j * SUB, SUB)]],
                    rows_v.at[b].at[pl.ds(j * SUB, SUB), :],
                    sg[b],
                )

        def drain_gathers(b):
            pltpu.make_async_copy(
                out_hbm.at[pl.ds(0, CHUNK)], rows_v.at[b], sg[b]
            ).wait()

        def drain_write(b):
            pltpu.make_async_copy(
                out_hbm.at[pl.ds(0, CHUNK)], rows_v.at[b], sw[b]
            ).wait()

        for b in range(NBUF - 1):
            fire_gathers(b, b)

        def round_body(r, carry):
            for b in range(NBUF):
                c = r * NBUF + b
                bf = (b + NBUF - 1) % NBUF
                f = c + NBUF - 1

                @pl.when(c >= 1)
                def _():
                    drain_write(bf)

                @pl.when(f < n_chunks)
                def _():
                    fire_gathers(f, bf)

                drain_gathers(b)
                pltpu.async_copy(
                    rows_v.at[b],
                    out_hbm.at[pl.ds(base + c * CHUNK, CHUNK)],
                    sw[b],
                )
            return carry

        lax.fori_loop(0, n_rounds, round_body, 0)
        drain_write((n_chunks - 1) % NBUF)

    return emb(xf, tpad)


def kernel(x, table):
    xf = x.reshape(-1).astype(jnp.int32)
    tpad = jnp.pad(table, ((0, 0), (0, PDIM - DIM)))
    out = _embed_sc(xf, tpad)
    return out[:, :DIM].reshape(x.shape + (DIM,))


# speedup vs baseline: 1.2401x; 1.0001x over previous
import functools

import jax
import jax.numpy as jnp
from jax import lax
from jax.experimental import pallas as pl
from jax.experimental.pallas import tpu as pltpu
from jax.experimental.pallas import tpu_sc as plsc

DIM = 64
NUM_CORES = 2
NUM_SUBCORES = 16
NUM_WORKERS = NUM_CORES * NUM_SUBCORES
PDIM = 128
SUB = 80
GATHERS = 5
CHUNK = SUB * GATHERS
NBUF = 2


@jax.jit
def _embed_sc(xf, tpad):
    B = xf.shape[0]
    b_per_w = B // NUM_WORKERS
    n_chunks = b_per_w // CHUNK
    n_rounds = n_chunks // NBUF
    assert b_per_w % CHUNK == 0 and n_chunks % NBUF == 0

    mesh = plsc.VectorSubcoreMesh(core_axis_name="c", subcore_axis_name="s")

    @functools.partial(
        pl.kernel,
        mesh=mesh,
        compiler_params=pltpu.CompilerParams(use_tc_tiling_on_sc=True),
        out_type=jax.ShapeDtypeStruct((B, PDIM), jnp.float32),
        scratch_types=[
            pltpu.VMEM((b_per_w,), jnp.int32),
            pltpu.VMEM((NBUF, CHUNK, PDIM), jnp.float32),
            pltpu.SemaphoreType.DMA,
            pltpu.SemaphoreType.DMA,
            pltpu.SemaphoreType.DMA,
            pltpu.SemaphoreType.DMA,
        ],
    )
    def emb(x_hbm, tab_hbm, out_hbm, idx_v, rows_v, sg0, sg1, sw0, sw1):
        sg = [sg0, sg1]
        sw = [sw0, sw1]
        wid = lax.axis_index("s") * NUM_CORES + lax.axis_index("c")
        base = wid * b_per_w

        pltpu.sync_copy(x_hbm.at[pl.ds(base, b_per_w)], idx_v)

        def fire_gathers(c, b):
            for j in range(GATHERS):
                pltpu.async_copy(
                    tab_hbm.at[idx_v.at[pl.ds(c * CHUNK + j * SUB, SUB)]],
                    rows_v.at[b].at[pl.ds(j * SUB, SUB), :],
                    sg[b],
                )

        def drain_gathers(b):
            pltpu.make_async_copy(
                out_hbm.at[pl.ds(0, CHUNK)], rows_v.at[b], sg[b]
            ).wait()

        def drain_write(b):
            pltpu.make_async_copy(
                out_hbm.at[pl.ds(0, CHUNK)], rows_v.at[b], sw[b]
            ).wait()

        for b in range(NBUF - 1):
            fire_gathers(b, b)

        def round_body(r, carry):
            for b in range(NBUF):
                c = r * NBUF + b
                bf = (b + NBUF - 1) % NBUF
                f = c + NBUF - 1

                @pl.when(c >= 1)
                def _():
                    drain_write(bf)

                @pl.when(f < n_chunks)
                def _():
                    fire_gathers(f, bf)

                drain_gathers(b)
                pltpu.async_copy(
                    rows_v.at[b],
                    out_hbm.at[pl.ds(base + c * CHUNK, CHUNK)],
                    sw[b],
                )
            return carry

        lax.fori_loop(0, n_rounds, round_body, 0)
        drain_write((n_chunks - 1) % NBUF)

    return emb(xf, tpad)


def kernel(x, table):
    xf = x.reshape(-1).astype(jnp.int32)
    tpad = jnp.pad(table, ((0, 0), (0, PDIM - DIM)))
    out = _embed_sc(xf, tpad)
    return out[:, :DIM].reshape(x.shape + (DIM,))


# speedup vs baseline: 1.2410x; 1.0007x over previous
import functools

import jax
import jax.numpy as jnp
from jax import lax
from jax.experimental import pallas as pl
from jax.experimental.pallas import tpu as pltpu
from jax.experimental.pallas import tpu_sc as plsc

DIM = 64
NUM_CORES = 2
NUM_SUBCORES = 16
NUM_WORKERS = NUM_CORES * NUM_SUBCORES
PDIM = 128
SUB = 80
GATHERS = 5
CHUNK = SUB * GATHERS
NBUF = 2


@jax.jit
def _embed_sc(xf, tpad):
    B = xf.shape[0]
    b_per_w = B // NUM_WORKERS
    n_chunks = b_per_w // CHUNK
    n_rounds = n_chunks // NBUF
    assert b_per_w % CHUNK == 0 and n_chunks % NBUF == 0

    mesh = plsc.VectorSubcoreMesh(core_axis_name="c", subcore_axis_name="s")

    @functools.partial(
        pl.kernel,
        mesh=mesh,
        compiler_params=pltpu.CompilerParams(use_tc_tiling_on_sc=True),
        out_type=jax.ShapeDtypeStruct((B, PDIM), jnp.float32),
        scratch_types=[
            pltpu.VMEM((b_per_w,), jnp.int32),
            pltpu.VMEM((NBUF, CHUNK, PDIM), jnp.float32),
            pltpu.SemaphoreType.DMA,
            pltpu.SemaphoreType.DMA,
            pltpu.SemaphoreType.DMA,
            pltpu.SemaphoreType.DMA,
        ],
    )
    def emb(x_hbm, tab_hbm, out_hbm, idx_v, rows_v, sg0, sg1, sw0, sw1):
        sg = [sg0, sg1]
        sw = [sw0, sw1]
        wid = lax.axis_index("s") * NUM_CORES + lax.axis_index("c")
        base = wid * b_per_w

        pltpu.sync_copy(x_hbm.at[pl.ds(base, b_per_w)], idx_v)

        def fire_gathers(c, b):
            for j in range(GATHERS):
                pltpu.async_copy(
                    tab_hbm.at[idx_v.at[pl.ds(c * CHUNK + j * SUB, SUB)]],
                    rows_v.at[b].at[pl.ds(j * SUB, SUB), :],
                    sg[b],
                )

        def drain_gathers(b):
            pltpu.make_async_copy(
                out_hbm.at[pl.ds(0, CHUNK)], rows_v.at[b], sg[b]
            ).wait()

        def drain_write(b):
            pltpu.make_async_copy(
                out_hbm.at[pl.ds(0, CHUNK)], rows_v.at[b], sw[b]
            ).wait()

        for b in range(NBUF - 1):
            fire_gathers(b, b)

        def round_body(r, carry):
            for b in range(NBUF):
                c = r * NBUF + b
                bf = (b + NBUF - 1) % NBUF
                f = c + NBUF - 1

                @pl.when(c >= 1)
                def _():
                    drain_write(bf)

                @pl.when(f < n_chunks)
                def _():
                    fire_gathers(f, bf)

                drain_gathers(b)
                pltpu.async_copy(
                    rows_v.at[b],
                    out_hbm.at[pl.ds(base + c * CHUNK, CHUNK)],
                    sw[b],
                )
            return carry

        lax.fori_loop(0, n_rounds, round_body, 0)
        drain_write((n_chunks - 1) % NBUF)

    return emb(xf, tpad)


def kernel(x, table):
    xf = x.reshape(-1).astype(jnp.int32)
    tpad = jnp.concatenate(
        [table, jnp.zeros((table.shape[0], PDIM - DIM), jnp.float32)], axis=1)
    out = _embed_sc(xf, tpad)
    return out[:, :DIM].reshape(x.shape + (DIM,))
